# baseline (speedup 1.0000x reference)
import jax
import jax.numpy as jnp
from jax import lax
from jax.experimental import pallas as pl
from jax.experimental.pallas import tpu as pltpu
from jax.experimental.pallas import tpu_sc as plsc

NC = 2
NS = 16
EB = 40
ROWS = 2000
NBUF = 4
GA = 3


def _seg_sum(table, src2d, dst2d, zeros_rows, zeros_deg, ones_deg, *,
             n_nodes, n_pad, width, with_deg):
    e_total = dst2d.shape[0] * dst2d.shape[1]
    rpt = n_pad // NS
    ept = e_total // NS
    assert ept % EB == 0 and n_pad % NS == 0
    nb = ept // EB
    assert nb % NBUF == 0 and nb > NBUF and NBUF == 4 and GA == 3

    mesh = plsc.VectorSubcoreMesh(core_axis_name="c", subcore_axis_name="s")

    out_type = [jax.ShapeDtypeStruct((NC, n_pad, width), jnp.float32)]
    scratch = [
        pltpu.VMEM((nb, EB), jnp.int32),
        pltpu.VMEM((NBUF, EB), jnp.int32),
        pltpu.VMEM((NBUF * EB, width), jnp.float32),
        pltpu.VMEM((rpt, width), jnp.float32),
        pltpu.VMEM_SHARED((n_pad, width), jnp.float32),
    ] + [pltpu.SemaphoreType.DMA] * (2 * NBUF + 3)
    if with_deg:
        out_type.append(jax.ShapeDtypeStruct((NC, n_pad, 16), jnp.float32))
        scratch += [
            pltpu.VMEM((EB, 16), jnp.float32),
            pltpu.VMEM((rpt, 16), jnp.float32),
            pltpu.VMEM_SHARED((n_pad, 16), jnp.float32),
        ] + [pltpu.SemaphoreType.DMA] * NBUF

    def body(table_h, src_h, dst_h, z_h, zd_h, one_h, *rest):
        acc_o = rest[0]
        k = 2 if with_deg else 1
        deg_o = rest[1] if with_deg else None
        (idx_v, dst_v, rows_v, wb_v, acc_sh) = rest[k:k + 5]
        sems = rest[k + 5:k + 5 + 2 * NBUF + 3]
        gsem = sems[:NBUF]
        ssem = sems[NBUF:2 * NBUF]
        xsem = sems[2 * NBUF:2 * NBUF + 2]
        isem = sems[2 * NBUF + 2]
        if with_deg:
            ones_v, wbd_v, deg_sh = rest[k + 8 + 2 * NBUF:k + 11 + 2 * NBUF]
            dsem = rest[k + 11 + 2 * NBUF:]
        else:
            ones_v = wbd_v = deg_sh = dsem = None

        c = lax.axis_index("c")
        s = lax.axis_index("s")
        r0 = s * rpt
        row0 = s * nb

        pltpu.async_copy(src_h.at[c, pl.ds(row0, nb)], idx_v, isem)

        pltpu.sync_copy(z_h, wb_v)
        pltpu.sync_copy(wb_v, acc_sh.at[pl.ds(r0, rpt)])
        if with_deg:
            pltpu.sync_copy(zd_h, wbd_v)
            pltpu.sync_copy(wbd_v, deg_sh.at[pl.ds(r0, rpt)])
            pltpu.sync_copy(one_h, ones_v)
        pltpu.make_async_copy(src_h.at[c, pl.ds(row0, nb)], idx_v, isem).wait()
        plsc.subcore_barrier()

        def buf(b):
            return rows_v.at[pl.ds(b * EB, EB)]

        def gather(i, b):
            pltpu.async_copy(table_h.at[idx_v.at[i]], buf(b), gsem[b])

        def gather_wait(b):
            pltpu.make_async_copy(table_h.at[idx_v.at[0]], buf(b),
                                  gsem[b]).wait()

        def dst_prefetch(i, p):
            pltpu.async_copy(dst_h.at[pl.ds(row0 + i, 2)],
                             dst_v.at[pl.ds(2 * p, 2)], xsem[p])

        def dst_wait(p):
            pltpu.make_async_copy(dst_h.at[pl.ds(row0, 2)],
                                  dst_v.at[pl.ds(2 * p, 2)], xsem[p]).wait()

        def scatter(r, b):
            pltpu.async_copy(buf(b), acc_sh.at[dst_v.at[r]], ssem[b],
                             add=True)
            if with_deg:
                pltpu.async_copy(ones_v, deg_sh.at[dst_v.at[r]], dsem[b],
                                 add=True)

        def scatter_wait(b):
            pltpu.make_async_copy(buf(b), acc_sh.at[dst_v.at[0]],
                                  ssem[b]).wait()
            if with_deg:
                pltpu.make_async_copy(ones_v, deg_sh.at[dst_v.at[0]],
                                      dsem[b]).wait()

        dst_prefetch(0, 0)
        for b in range(GA):
            gather(b, b)

        def outer(ii, carry):
            for r in range(NBUF):
                i = ii * NBUF + r
                gather_wait(r)
                if r % 2 == 0:
                    dst_wait((r // 2) % 2)
                scatter(r, r)
                tgt = (r + GA) % NBUF

                @pl.when(i >= NBUF - GA)
                def _():
                    scatter_wait(tgt)

                if r % 2 == 0:
                    @pl.when(i + 2 < nb)
                    def _():
                        dst_prefetch(i + 2, (r // 2 + 1) % 2)

                @pl.when(i + GA < nb)
                def _():
                    gather(i + GA, tgt)
            return carry

        lax.fori_loop(0, nb // NBUF, outer, 0)
        for j in range(nb - (NBUF - GA), nb):
            scatter_wait(j % NBUF)
        plsc.subcore_barrier()

        pltpu.sync_copy(acc_sh.at[pl.ds(r0, rpt)], wb_v)
        pltpu.sync_copy(wb_v, acc_o.at[c, pl.ds(r0, rpt)])
        if with_deg:
            pltpu.sync_copy(deg_sh.at[pl.ds(r0, rpt)], wbd_v)
            pltpu.sync_copy(wbd_v, deg_o.at[c, pl.ds(r0, rpt)])

    return pl.kernel(
        body, out_type=out_type, mesh=mesh, scratch_types=scratch,
        compiler_params=pltpu.CompilerParams(use_tc_tiling_on_sc=False),
    )(table, src2d, dst2d, zeros_rows, zeros_deg, ones_deg)


def _inv_deg(deg_ref):
    d = jnp.max(deg_ref[0], axis=1, keepdims=True)
    return 1.0 / jnp.maximum(d, 1.0)


def _accumulate_stats(i, h, st_ref):
    @pl.when(i == 0)
    def _():
        st_ref[...] = jnp.zeros_like(st_ref)
    st_ref[0:1, :] += jnp.sum(h, axis=0, keepdims=True)
    st_ref[1:2, :] += jnp.sum(h * h, axis=0, keepdims=True)


def _sage1_body(acc_ref, deg_ref, xr_ref, wl_ref, h_ref, st_ref):
    i = pl.program_id(0)
    inv = _inv_deg(deg_ref)
    mean = jnp.concatenate([acc_ref[0], acc_ref[1]], axis=1) * inv
    h = (jnp.dot(mean, wl_ref[...], preferred_element_type=jnp.float32)
         + xr_ref[...])
    h_ref[...] = h
    _accumulate_stats(i, h, st_ref)


def _root_body(x_ref, wr_ref, bl_ref, o_ref):
    o_ref[...] = (jnp.dot(x_ref[...], wr_ref[...],
                          preferred_element_type=jnp.float32) + bl_ref[...])


def _combine_body(acc_ref, deg_ref, r_ref, bl_ref, h_ref, st_ref):
    i = pl.program_id(0)
    inv = _inv_deg(deg_ref)
    h = (jnp.concatenate([acc_ref[0], acc_ref[1]], axis=1) * inv
         + r_ref[...] + bl_ref[...])
    h_ref[...] = h
    _accumulate_stats(i, h, st_ref)


def _bn_project_body(h_ref, st_ref, g_ref, be_ref, wls_ref,
                     y_ref, yf_ref, *, n_nodes):
    mu = st_ref[0:1, :] * (1.0 / n_nodes)
    var = st_ref[1:2, :] * (1.0 / n_nodes) - mu * mu
    scale = g_ref[...] * lax.rsqrt(var + 1e-5)
    shift = be_ref[...] - mu * scale
    y = jnp.maximum(h_ref[...] * scale + shift, 0.0)
    y_ref[0] = jnp.dot(y, wls_ref[0], preferred_element_type=jnp.float32)
    y_ref[1] = jnp.dot(y, wls_ref[1], preferred_element_type=jnp.float32)
    yf_ref[...] = y


def _final_body(acc_ref, deg_ref, r_ref, bl_ref, wc_ref, bc_ref, o_ref):
    inv = _inv_deg(deg_ref)
    h = (jnp.concatenate([acc_ref[0], acc_ref[1]], axis=1) * inv
         + r_ref[...] + bl_ref[...])
    h = jnp.maximum(h, 0.0)
    logits = jnp.dot(h, wc_ref[...], preferred_element_type=jnp.float32) + bc_ref[...]
    o_ref[...] = 1.0 / (1.0 + jnp.exp(-logits))


def _full(shape):
    return pl.BlockSpec(shape, lambda i: (0,) * len(shape))


def _rows3(nlead, width):
    return pl.BlockSpec((nlead, ROWS, width), lambda i: (0, i, 0))


def _rows2(width):
    return pl.BlockSpec((ROWS, width), lambda i: (i, 0))


def kernel(x, edge_index, Wl1, bl1, Wr1, g1, be1, Wl2, bl2, Wr2, g2, be2,
           Wl3, bl3, Wr3, Wc, bc):
    n, d_in = x.shape
    e = edge_index.shape[1]
    h1w = Wl1.shape[1]
    h2w = Wl2.shape[1]
    h3w = Wl3.shape[1]
    grid = n // ROWS

    src = edge_index[0]
    dst = edge_index[1]
    src2d = jnp.stack([src, src + n]).reshape(NC, e // EB, EB)
    dst2d = dst.reshape(e // EB, EB)
    n_pad = ((n + NS - 1) // NS) * NS
    rpt = n_pad // NS

    zeros_rows = jnp.zeros((rpt, d_in // 2), jnp.float32)
    zeros_rows3 = jnp.zeros((rpt, h3w // 2), jnp.float32)
    zeros_deg = jnp.zeros((rpt, 16), jnp.float32)
    ones_deg = jnp.ones((EB, 16), jnp.float32)

    wl2s = jnp.stack([Wl2[:, :h2w // 2], Wl2[:, h2w // 2:]])
    wl3s = jnp.stack([Wl3[:, :h3w // 2], Wl3[:, h3w // 2:]])

    x2 = jnp.concatenate([x[:, :d_in // 2], x[:, d_in // 2:]], axis=0)
    acc1, deg = _seg_sum(x2, src2d, dst2d, zeros_rows, zeros_deg, ones_deg,
                         n_nodes=n, n_pad=n_pad, width=d_in // 2,
                         with_deg=True)

    def root(xin, wr, bl, din, dout):
        return pl.pallas_call(
            _root_body,
            grid=(grid,),
            in_specs=[_rows2(din), _full((din, dout)), _full((1, dout))],
            out_specs=_rows2(dout),
            out_shape=jax.ShapeDtypeStruct((n, dout), jnp.float32),
        )(xin, wr, bl.reshape(1, -1))

    xr1 = root(x, Wr1, bl1, d_in, h1w)

    h1, st1 = pl.pallas_call(
        _sage1_body,
        grid=(grid,),
        in_specs=[_rows3(NC, d_in // 2), _rows3(NC, 16), _rows2(h1w),
                  _full((d_in, h1w))],
        out_specs=[_rows2(h1w), _full((8, h1w))],
        out_shape=[jax.ShapeDtypeStruct((n, h1w), jnp.float32),
                   jax.ShapeDtypeStruct((8, h1w), jnp.float32)],
    )(acc1, deg, xr1, Wl1)

    def bn_project(h, st, g, be, wls, hw, ow):
        return pl.pallas_call(
            lambda *a: _bn_project_body(*a, n_nodes=n),
            grid=(grid,),
            in_specs=[_rows2(hw), _full((8, hw)), _full((1, hw)),
                      _full((1, hw)), _full(wls.shape)],
            out_specs=[_rows3(2, ow // 2), _rows2(hw)],
            out_shape=[jax.ShapeDtypeStruct((2, n, ow // 2), jnp.float32),
                       jax.ShapeDtypeStruct((n, hw), jnp.float32)],
        )(h, st, g.reshape(1, -1), be.reshape(1, -1), wls)

    y2, y1f = bn_project(h1, st1, g1, be1, wl2s, h1w, h2w)

    acc2 = _seg_sum(y2.reshape(2 * n, h2w // 2), src2d, dst2d,
                    zeros_rows, zeros_deg, ones_deg,
                    n_nodes=n, n_pad=n_pad, width=h2w // 2,
                    with_deg=False)[0]
    r2 = root(y1f, Wr2, bl2, h1w, h2w)

    h2, st2 = pl.pallas_call(
        _combine_body,
        grid=(grid,),
        in_specs=[_rows3(NC, h2w // 2), _rows3(NC, 16), _rows2(h2w),
                  _full((1, h2w))],
        out_specs=[_rows2(h2w), _full((8, h2w))],
        out_shape=[jax.ShapeDtypeStruct((n, h2w), jnp.float32),
                   jax.ShapeDtypeStruct((8, h2w), jnp.float32)],
    )(acc2, deg, r2, jnp.zeros((1, h2w), jnp.float32))

    y3, y2f = bn_project(h2, st2, g2, be2, wl3s, h2w, h3w)

    acc3 = _seg_sum(y3.reshape(2 * n, h3w // 2), src2d, dst2d,
                    zeros_rows3, zeros_deg, ones_deg,
                    n_nodes=n, n_pad=n_pad, width=h3w // 2,
                    with_deg=False)[0]
    r3 = root(y2f, Wr3, bl3, h2w, h3w)

    wc_pad = jnp.pad(Wc, ((0, 0), (0, 128 - Wc.shape[1])))
    out_full = pl.pallas_call(
        _final_body,
        grid=(grid,),
        in_specs=[_rows3(NC, h3w // 2), _rows3(NC, 16), _rows2(h3w),
                  _full((1, h3w)), _full((h3w, 128)), _full((1, 1))],
        out_specs=_rows2(128),
        out_shape=jax.ShapeDtypeStruct((n, 128), jnp.float32),
    )(acc3, deg, r3, jnp.zeros((1, h3w), jnp.float32), wc_pad,
      bc.reshape(1, 1))

    return out_full[:, 0]


# speedup vs baseline: 1.0292x; 1.0292x over previous
import jax
import jax.numpy as jnp
from jax import lax
from jax.experimental import pallas as pl
from jax.experimental.pallas import tpu as pltpu
from jax.experimental.pallas import tpu_sc as plsc

NC = 2
NS = 16
EB = 40
ROWS = 2000
NBUF = 4
GA = 3


def _seg_sum(table, src2d, dst2d, zeros_rows, zeros_deg, ones_deg, *,
             n_nodes, n_pad, width, with_deg):
    e_total = dst2d.shape[0] * dst2d.shape[1]
    rpt = n_pad // NS
    ept = e_total // NS
    assert ept % EB == 0 and n_pad % NS == 0
    nb = ept // EB
    assert nb % NBUF == 0 and nb > NBUF and NBUF == 4 and GA == 3

    mesh = plsc.VectorSubcoreMesh(core_axis_name="c", subcore_axis_name="s")

    out_type = [jax.ShapeDtypeStruct((NC, n_pad, width), jnp.float32)]
    scratch = [
        pltpu.VMEM((nb, EB), jnp.int32),
        pltpu.VMEM((NBUF, EB), jnp.int32),
        pltpu.VMEM((NBUF * EB, width), jnp.float32),
        pltpu.VMEM((rpt, width), jnp.float32),
        pltpu.VMEM_SHARED((n_pad, width), jnp.float32),
    ] + [pltpu.SemaphoreType.DMA] * (3 * NBUF + 1)
    if with_deg:
        out_type.append(jax.ShapeDtypeStruct((NC, n_pad, 16), jnp.float32))
        scratch += [
            pltpu.VMEM((EB, 16), jnp.float32),
            pltpu.VMEM((rpt, 16), jnp.float32),
            pltpu.VMEM_SHARED((n_pad, 16), jnp.float32),
        ] + [pltpu.SemaphoreType.DMA] * NBUF

    def body(table_h, src_h, dst_h, z_h, zd_h, one_h, *rest):
        acc_o = rest[0]
        k = 2 if with_deg else 1
        deg_o = rest[1] if with_deg else None
        (idx_v, dst_v, rows_v, wb_v, acc_sh) = rest[k:k + 5]
        sems = rest[k + 5:k + 5 + 3 * NBUF + 1]
        gsem = sems[:NBUF]
        ssem = sems[NBUF:2 * NBUF]
        xsem = sems[2 * NBUF:3 * NBUF]
        isem = sems[3 * NBUF]
        if with_deg:
            ones_v, wbd_v, deg_sh = rest[k + 6 + 3 * NBUF:k + 9 + 3 * NBUF]
            dsem = rest[k + 9 + 3 * NBUF:]
        else:
            ones_v = wbd_v = deg_sh = dsem = None

        c = lax.axis_index("c")
        s = lax.axis_index("s")
        r0 = s * rpt
        row0 = s * nb

        pltpu.async_copy(src_h.at[c, pl.ds(row0, nb)], idx_v, isem)

        pltpu.sync_copy(z_h, wb_v)
        pltpu.sync_copy(wb_v, acc_sh.at[pl.ds(r0, rpt)])
        if with_deg:
            pltpu.sync_copy(zd_h, wbd_v)
            pltpu.sync_copy(wbd_v, deg_sh.at[pl.ds(r0, rpt)])
            pltpu.sync_copy(one_h, ones_v)
        pltpu.make_async_copy(src_h.at[c, pl.ds(row0, nb)], idx_v, isem).wait()
        plsc.subcore_barrier()

        def buf(b):
            return rows_v.at[pl.ds(b * EB, EB)]

        def gather(i, b):
            pltpu.async_copy(table_h.at[idx_v.at[i]], buf(b), gsem[b])

        def gather_wait(b):
            pltpu.make_async_copy(table_h.at[idx_v.at[0]], buf(b),
                                  gsem[b]).wait()

        def dst_prefetch(i, b):
            pltpu.async_copy(dst_h.at[pl.ds(row0 + i, 1)],
                             dst_v.at[pl.ds(b, 1)], xsem[b])

        def dst_wait(b):
            pltpu.make_async_copy(dst_h.at[pl.ds(row0, 1)],
                                  dst_v.at[pl.ds(b, 1)], xsem[b]).wait()

        def scatter(r, b):
            pltpu.async_copy(buf(b), acc_sh.at[dst_v.at[r]], ssem[b],
                             add=True)
            if with_deg:
                pltpu.async_copy(ones_v, deg_sh.at[dst_v.at[r]], dsem[b],
                                 add=True)

        def scatter_wait(b):
            pltpu.make_async_copy(buf(b), acc_sh.at[dst_v.at[0]],
                                  ssem[b]).wait()
            if with_deg:
                pltpu.make_async_copy(ones_v, deg_sh.at[dst_v.at[0]],
                                      dsem[b]).wait()

        for b in range(GA):
            dst_prefetch(b, b)
            gather(b, b)

        def outer(ii, carry):
            for r in range(NBUF):
                i = ii * NBUF + r
                gather_wait(r)
                dst_wait(r)
                scatter(r, r)
                tgt = (r + GA) % NBUF

                @pl.when(i >= NBUF - GA)
                def _():
                    scatter_wait(tgt)

                @pl.when(i + GA < nb)
                def _():
                    dst_prefetch(i + GA, tgt)
                    gather(i + GA, tgt)
            return carry

        lax.fori_loop(0, nb // NBUF, outer, 0)
        for j in range(nb - (NBUF - GA), nb):
            scatter_wait(j % NBUF)
        plsc.subcore_barrier()

        pltpu.sync_copy(acc_sh.at[pl.ds(r0, rpt)], wb_v)
        pltpu.sync_copy(wb_v, acc_o.at[c, pl.ds(r0, rpt)])
        if with_deg:
            pltpu.sync_copy(deg_sh.at[pl.ds(r0, rpt)], wbd_v)
            pltpu.sync_copy(wbd_v, deg_o.at[c, pl.ds(r0, rpt)])

    return pl.kernel(
        body, out_type=out_type, mesh=mesh, scratch_types=scratch,
        compiler_params=pltpu.CompilerParams(use_tc_tiling_on_sc=False),
    )(table, src2d, dst2d, zeros_rows, zeros_deg, ones_deg)


def _inv_deg(deg_ref):
    d = jnp.max(deg_ref[0], axis=1, keepdims=True)
    return 1.0 / jnp.maximum(d, 1.0)


def _accumulate_stats(i, h, st_ref):
    @pl.when(i == 0)
    def _():
        st_ref[...] = jnp.zeros_like(st_ref)
    st_ref[0:1, :] += jnp.sum(h, axis=0, keepdims=True)
    st_ref[1:2, :] += jnp.sum(h * h, axis=0, keepdims=True)


def _sage1_body(acc_ref, deg_ref, x_ref, wl_ref, bl_ref, wr_ref, h_ref, st_ref):
    i = pl.program_id(0)
    inv = _inv_deg(deg_ref)
    mean = jnp.concatenate([acc_ref[0], acc_ref[1]], axis=1) * inv
    h = (jnp.dot(mean, wl_ref[...], preferred_element_type=jnp.float32)
         + jnp.dot(x_ref[...], wr_ref[...], preferred_element_type=jnp.float32)
         + bl_ref[...])
    h_ref[...] = h
    _accumulate_stats(i, h, st_ref)


def _combine_body(acc_ref, deg_ref, r_ref, bl_ref, h_ref, st_ref):
    i = pl.program_id(0)
    inv = _inv_deg(deg_ref)
    h = (jnp.concatenate([acc_ref[0], acc_ref[1]], axis=1) * inv
         + r_ref[...] + bl_ref[...])
    h_ref[...] = h
    _accumulate_stats(i, h, st_ref)


def _bn_project_body(h_ref, st_ref, g_ref, be_ref, wls_ref, wr_ref,
                     y_ref, r_ref, *, n_nodes):
    mu = st_ref[0:1, :] * (1.0 / n_nodes)
    var = st_ref[1:2, :] * (1.0 / n_nodes) - mu * mu
    scale = g_ref[...] * lax.rsqrt(var + 1e-5)
    shift = be_ref[...] - mu * scale
    y = jnp.maximum(h_ref[...] * scale + shift, 0.0)
    y_ref[0] = jnp.dot(y, wls_ref[0], preferred_element_type=jnp.float32)
    y_ref[1] = jnp.dot(y, wls_ref[1], preferred_element_type=jnp.float32)
    r_ref[...] = jnp.dot(y, wr_ref[...], preferred_element_type=jnp.float32)


def _final_body(acc_ref, deg_ref, r_ref, bl_ref, wc_ref, bc_ref, o_ref):
    inv = _inv_deg(deg_ref)
    h = (jnp.concatenate([acc_ref[0], acc_ref[1]], axis=1) * inv
         + r_ref[...] + bl_ref[...])
    h = jnp.maximum(h, 0.0)
    logits = jnp.dot(h, wc_ref[...], preferred_element_type=jnp.float32) + bc_ref[...]
    o_ref[...] = 1.0 / (1.0 + jnp.exp(-logits))


def _full(shape):
    return pl.BlockSpec(shape, lambda i: (0,) * len(shape))


def _rows3(nlead, width):
    return pl.BlockSpec((nlead, ROWS, width), lambda i: (0, i, 0))


def _rows2(width):
    return pl.BlockSpec((ROWS, width), lambda i: (i, 0))


def kernel(x, edge_index, Wl1, bl1, Wr1, g1, be1, Wl2, bl2, Wr2, g2, be2,
           Wl3, bl3, Wr3, Wc, bc):
    n, d_in = x.shape
    e = edge_index.shape[1]
    h1w = Wl1.shape[1]
    h2w = Wl2.shape[1]
    h3w = Wl3.shape[1]
    grid = n // ROWS

    src = edge_index[0]
    dst = edge_index[1]
    src2d = jnp.stack([src, src + n]).reshape(NC, e // EB, EB)
    dst2d = dst.reshape(e // EB, EB)
    n_pad = ((n + NS - 1) // NS) * NS
    rpt = n_pad // NS

    zeros_rows = jnp.zeros((rpt, d_in // 2), jnp.float32)
    zeros_rows3 = jnp.zeros((rpt, h3w // 2), jnp.float32)
    zeros_deg = jnp.zeros((rpt, 16), jnp.float32)
    ones_deg = jnp.ones((EB, 16), jnp.float32)

    wl2s = jnp.stack([Wl2[:, :h2w // 2], Wl2[:, h2w // 2:]])
    wl3s = jnp.stack([Wl3[:, :h3w // 2], Wl3[:, h3w // 2:]])

    x2 = jnp.concatenate([x[:, :d_in // 2], x[:, d_in // 2:]], axis=0)
    acc1, deg = _seg_sum(x2, src2d, dst2d, zeros_rows, zeros_deg, ones_deg,
                         n_nodes=n, n_pad=n_pad, width=d_in // 2,
                         with_deg=True)

    h1, st1 = pl.pallas_call(
        _sage1_body,
        grid=(grid,),
        in_specs=[_rows3(NC, d_in // 2), _rows3(NC, 16), _rows2(d_in),
                  _full((d_in, h1w)), _full((1, h1w)), _full((d_in, h1w))],
        out_specs=[_rows2(h1w), _full((8, h1w))],
        out_shape=[jax.ShapeDtypeStruct((n, h1w), jnp.float32),
                   jax.ShapeDtypeStruct((8, h1w), jnp.float32)],
    )(acc1, deg, x, Wl1, bl1.reshape(1, -1), Wr1)

    def bn_project(h, st, g, be, wls, wr, hw, ow):
        return pl.pallas_call(
            lambda *a: _bn_project_body(*a, n_nodes=n),
            grid=(grid,),
            in_specs=[_rows2(hw), _full((8, hw)), _full((1, hw)),
                      _full((1, hw)), _full(wls.shape), _full(wr.shape)],
            out_specs=[_rows3(2, ow // 2), _rows2(ow)],
            out_shape=[jax.ShapeDtypeStruct((2, n, ow // 2), jnp.float32),
                       jax.ShapeDtypeStruct((n, ow), jnp.float32)],
        )(h, st, g.reshape(1, -1), be.reshape(1, -1), wls, wr)

    y2, r2 = bn_project(h1, st1, g1, be1, wl2s, Wr2, h1w, h2w)

    acc2 = _seg_sum(y2.reshape(2 * n, h2w // 2), src2d, dst2d,
                    zeros_rows, zeros_deg, ones_deg,
                    n_nodes=n, n_pad=n_pad, width=h2w // 2,
                    with_deg=False)[0]

    h2, st2 = pl.pallas_call(
        _combine_body,
        grid=(grid,),
        in_specs=[_rows3(NC, h2w // 2), _rows3(NC, 16), _rows2(h2w),
                  _full((1, h2w))],
        out_specs=[_rows2(h2w), _full((8, h2w))],
        out_shape=[jax.ShapeDtypeStruct((n, h2w), jnp.float32),
                   jax.ShapeDtypeStruct((8, h2w), jnp.float32)],
    )(acc2, deg, r2, bl2.reshape(1, -1))

    y3, r3 = bn_project(h2, st2, g2, be2, wl3s, Wr3, h2w, h3w)

    acc3 = _seg_sum(y3.reshape(2 * n, h3w // 2), src2d, dst2d,
                    zeros_rows3, zeros_deg, ones_deg,
                    n_nodes=n, n_pad=n_pad, width=h3w // 2,
                    with_deg=False)[0]

    wc_pad = jnp.pad(Wc, ((0, 0), (0, 128 - Wc.shape[1])))
    out_full = pl.pallas_call(
        _final_body,
        grid=(grid,),
        in_specs=[_rows3(NC, h3w // 2), _rows3(NC, 16), _rows2(h3w),
                  _full((1, h3w)), _full((h3w, 128)), _full((1, 1))],
        out_specs=_rows2(128),
        out_shape=jax.ShapeDtypeStruct((n, 128), jnp.float32),
    )(acc3, deg, r3, bl3.reshape(1, -1), wc_pad, bc.reshape(1, 1))

    return out_full[:, 0]
